# baseline (speedup 1.0000x reference)
import functools

import jax
import jax.numpy as jnp
from jax.experimental import pallas as pl
from jax.experimental.pallas import tpu as pltpu

L, B, D, H = 1024, 8, 256, 256
CH = 128


def _gates(gx, gh, h_prev):
    r = jax.nn.sigmoid(gx[:, :H] + gh[:, :H])
    z = jax.nn.sigmoid(gx[:, H:2 * H] + gh[:, H:2 * H])
    n = jnp.tanh(gx[:, 2 * H:] + r * gh[:, 2 * H:])
    return (1.0 - z) * n + z * h_prev


def _smask(conds):
    cols = [jnp.broadcast_to(jnp.where(c, 1.0, 0.0), (1, 1)) for c in conds]
    return jnp.concatenate(cols, axis=0)


def _rnn_kernel(heads_ref, x_ref, wdt_ref, udt_ref, bdt_ref, wtd_ref,
                utd_ref, btd_ref, out_ref, child_sum_ref, htd_ref,
                gxdt_ref, gxtd_ref):
    child_sum_ref[...] = jnp.zeros((L, B, H), jnp.float32)

    def chunk(c, carry):
        base_dt = L - CH * (c + 1)
        xd = x_ref[pl.ds(base_dt, CH)].reshape(CH * B, D)
        gd = jnp.dot(xd, wdt_ref[...], preferred_element_type=jnp.float32)
        gxdt_ref[...] = (gd + bdt_ref[...]).reshape(CH, B, 3 * H)
        xt = x_ref[pl.ds(CH * c, CH)].reshape(CH * B, D)
        gt = jnp.dot(xt, wtd_ref[...], preferred_element_type=jnp.float32)
        gxtd_ref[...] = (gt + btd_ref[...]).reshape(CH, B, 3 * H)
        return jax.lax.fori_loop(0, CH, step_of(c), carry, unroll=16)

    def step_of(c):
        return lambda s, carry: step(c * CH + s, s, carry)

    def step(t, s, carry):
        h_sum, h_par = carry

        i = L - 1 - t
        gx = gxdt_ref[pl.ds(CH - 1 - s, 1)].reshape(B, 3 * H)
        gh = jnp.dot(h_sum.astype(jnp.bfloat16), udt_ref[...],
                     preferred_element_type=jnp.float32)
        h_dt = _gates(gx, gh, h_sum)
        out_ref[:, pl.ds(i, 1), 0:H] = h_dt.reshape(B, 1, H)

        inext = jnp.maximum(i - 1, 0)
        h_sum_next = child_sum_ref[pl.ds(inext, 1)].reshape(B, H)

        hbs = [heads_ref[b, i] for b in range(B)]
        u = _smask([hb == i - 1 for hb in hbs])
        g = jnp.where(i > 0, 1.0, 0.0)
        h_sum_next = h_sum_next + h_dt * (g * u)
        upd = h_dt * (g * (1.0 - u))
        loaded = [child_sum_ref[pl.ds(hbs[b], 1), b, :] for b in range(B)]
        for b in range(B):
            child_sum_ref[pl.ds(hbs[b], 1), b, :] = (
                loaded[b] + upd[b:b + 1, :])

        i2 = t
        gx2 = gxtd_ref[pl.ds(s, 1)].reshape(B, 3 * H)
        gh2 = jnp.dot(h_par.astype(jnp.bfloat16), utd_ref[...],
                      preferred_element_type=jnp.float32)
        h_td = _gates(gx2, gh2, h_par)
        out_ref[:, pl.ds(i2, 1), H:2 * H] = h_td.reshape(B, 1, H)

        tn = jnp.minimum(t + 1, L - 1)
        hb2s = [heads_ref[b, tn] for b in range(B)]
        rows = [htd_ref[pl.ds(hb2s[b], 1), b, :] for b in range(B)]
        gath = jnp.concatenate(rows, axis=0)
        u2 = _smask([hb2 == i2 for hb2 in hb2s])
        h_par_next = jnp.where(u2 > 0.0, h_td, gath)
        htd_ref[pl.ds(i2, 1)] = h_td.reshape(1, B, H)

        return (h_sum_next, h_par_next)

    zero = jnp.zeros((B, H), jnp.float32)
    jax.lax.fori_loop(0, L // CH, chunk, (zero, zero))


@functools.partial(jax.jit, static_argnames=())
def kernel(input, heads, W_dt, U_dt, b_dt, W_td, U_td, b_td):
    heads_i32 = heads.astype(jnp.int32)
    outputs = pl.pallas_call(
        _rnn_kernel,
        out_shape=jax.ShapeDtypeStruct((B, L, 2 * H), jnp.float32),
        in_specs=[
            pl.BlockSpec(memory_space=pltpu.SMEM),
            pl.BlockSpec(memory_space=pltpu.VMEM),
            pl.BlockSpec(memory_space=pltpu.VMEM),
            pl.BlockSpec(memory_space=pltpu.VMEM),
            pl.BlockSpec(memory_space=pltpu.VMEM),
            pl.BlockSpec(memory_space=pltpu.VMEM),
            pl.BlockSpec(memory_space=pltpu.VMEM),
            pl.BlockSpec(memory_space=pltpu.VMEM),
        ],
        out_specs=pl.BlockSpec(memory_space=pltpu.VMEM),
        scratch_shapes=[
            pltpu.VMEM((L, B, H), jnp.float32),
            pltpu.VMEM((L, B, H), jnp.float32),
            pltpu.VMEM((CH, B, 3 * H), jnp.float32),
            pltpu.VMEM((CH, B, 3 * H), jnp.float32),
        ],
    )(heads_i32, input, W_dt, U_dt.astype(jnp.bfloat16),
      b_dt.reshape(1, 3 * H), W_td, U_td.astype(jnp.bfloat16),
      b_td.reshape(1, 3 * H))
    output_t = outputs[:, 0, :][None, :, :]
    return outputs, output_t


# speedup vs baseline: 1.0363x; 1.0363x over previous
import functools

import jax
import jax.numpy as jnp
from jax.experimental import pallas as pl
from jax.experimental.pallas import tpu as pltpu

L, B, D, H = 1024, 8, 256, 256
CH = 128


def _gates(gx, gh, h_prev):
    r = jax.nn.sigmoid(gx[:, :H] + gh[:, :H])
    z = jax.nn.sigmoid(gx[:, H:2 * H] + gh[:, H:2 * H])
    n = jnp.tanh(gx[:, 2 * H:] + r * gh[:, 2 * H:])
    return (1.0 - z) * n + z * h_prev


def _smask(conds):
    cols = [jnp.broadcast_to(jnp.where(c, 1.0, 0.0), (1, 1)) for c in conds]
    return jnp.concatenate(cols, axis=0)


def _rnn_kernel(heads_ref, x_ref, wdt_ref, udt_ref, bdt_ref, wtd_ref,
                utd_ref, btd_ref, out_ref, child_sum_ref, htd_ref,
                gxdt_ref, gxtd_ref):
    child_sum_ref[...] = jnp.zeros((L, B, H), jnp.float32)

    def chunk(c, carry):
        base_dt = L - CH * (c + 1)
        xd = x_ref[pl.ds(base_dt, CH)].reshape(CH * B, D)
        gd = jnp.dot(xd, wdt_ref[...], preferred_element_type=jnp.float32)
        gxdt_ref[...] = (gd + bdt_ref[...]).reshape(CH, B, 3 * H)
        xt = x_ref[pl.ds(CH * c, CH)].reshape(CH * B, D)
        gt = jnp.dot(xt, wtd_ref[...], preferred_element_type=jnp.float32)
        gxtd_ref[...] = (gt + btd_ref[...]).reshape(CH, B, 3 * H)
        return jax.lax.fori_loop(0, CH, step_of(c), carry, unroll=8)

    def step_of(c):
        return lambda s, carry: step(c * CH + s, s, carry)

    def step(t, s, carry):
        h_sum, h_par = carry

        i = L - 1 - t
        gx = gxdt_ref[pl.ds(CH - 1 - s, 1)].reshape(B, 3 * H)
        gh = jnp.dot(h_sum.astype(jnp.bfloat16), udt_ref[...],
                     preferred_element_type=jnp.float32)
        h_dt = _gates(gx, gh, h_sum)
        out_ref[:, pl.ds(i, 1), 0:H] = h_dt.reshape(B, 1, H)

        inext = jnp.maximum(i - 1, 0)
        h_sum_next = child_sum_ref[pl.ds(inext, 1)].reshape(B, H)

        hbs = [heads_ref[b, i] for b in range(B)]
        u = _smask([hb == i - 1 for hb in hbs])
        g = jnp.where(i > 0, 1.0, 0.0)
        h_sum_next = h_sum_next + h_dt * (g * u)
        upd = h_dt * (g * (1.0 - u))
        loaded = [child_sum_ref[pl.ds(hbs[b], 1), b, :] for b in range(B)]
        for b in range(B):
            child_sum_ref[pl.ds(hbs[b], 1), b, :] = (
                loaded[b] + upd[b:b + 1, :])

        i2 = t
        gx2 = gxtd_ref[pl.ds(s, 1)].reshape(B, 3 * H)
        gh2 = jnp.dot(h_par.astype(jnp.bfloat16), utd_ref[...],
                      preferred_element_type=jnp.float32)
        h_td = _gates(gx2, gh2, h_par)
        out_ref[:, pl.ds(i2, 1), H:2 * H] = h_td.reshape(B, 1, H)

        tn = jnp.minimum(t + 1, L - 1)
        hb2s = [heads_ref[b, tn] for b in range(B)]
        rows = [htd_ref[pl.ds(hb2s[b], 1), b, :] for b in range(B)]
        gath = jnp.concatenate(rows, axis=0)
        u2 = _smask([hb2 == i2 for hb2 in hb2s])
        h_par_next = jnp.where(u2 > 0.0, h_td, gath)
        htd_ref[pl.ds(i2, 1)] = h_td.reshape(1, B, H)

        return (h_sum_next, h_par_next)

    zero = jnp.zeros((B, H), jnp.float32)
    jax.lax.fori_loop(0, L // CH, chunk, (zero, zero))


@functools.partial(jax.jit, static_argnames=())
def kernel(input, heads, W_dt, U_dt, b_dt, W_td, U_td, b_td):
    heads_i32 = heads.astype(jnp.int32)
    outputs = pl.pallas_call(
        _rnn_kernel,
        out_shape=jax.ShapeDtypeStruct((B, L, 2 * H), jnp.float32),
        in_specs=[
            pl.BlockSpec(memory_space=pltpu.SMEM),
            pl.BlockSpec(memory_space=pltpu.VMEM),
            pl.BlockSpec(memory_space=pltpu.VMEM),
            pl.BlockSpec(memory_space=pltpu.VMEM),
            pl.BlockSpec(memory_space=pltpu.VMEM),
            pl.BlockSpec(memory_space=pltpu.VMEM),
            pl.BlockSpec(memory_space=pltpu.VMEM),
            pl.BlockSpec(memory_space=pltpu.VMEM),
        ],
        out_specs=pl.BlockSpec(memory_space=pltpu.VMEM),
        scratch_shapes=[
            pltpu.VMEM((L, B, H), jnp.float32),
            pltpu.VMEM((L, B, H), jnp.float32),
            pltpu.VMEM((CH, B, 3 * H), jnp.float32),
            pltpu.VMEM((CH, B, 3 * H), jnp.float32),
        ],
    )(heads_i32, input, W_dt, U_dt.astype(jnp.bfloat16),
      b_dt.reshape(1, 3 * H), W_td, U_td.astype(jnp.bfloat16),
      b_td.reshape(1, 3 * H))
    output_t = outputs[:, 0, :][None, :, :]
    return outputs, output_t
